# baseline (speedup 1.0000x reference)
import functools

import jax
import jax.numpy as jnp
from jax import lax
from jax.experimental import pallas as pl
from jax.experimental.pallas import tpu as pltpu
from jax.experimental.pallas import tpu_sc as plsc

NUM_CLASSES = 100000
FEAT_DIM = 64
BATCH = 16384

NC, NS, L = 2, 16, 16
NW = NC * NS
BPW = BATCH // NW
GROUPS = BPW // L
CH = 128
NCH = BPW // CH

_mesh = plsc.VectorSubcoreMesh(core_axis_name="c", subcore_axis_name="s")


@functools.partial(
    pl.kernel,
    out_type=jax.ShapeDtypeStruct((NW, L), jnp.float32),
    mesh=_mesh,
    scratch_types=[
        pltpu.VMEM((BPW,), jnp.int32),
        pltpu.VMEM((2, CH, FEAT_DIM), jnp.float32),
        pltpu.VMEM((2, CH, FEAT_DIM), jnp.float32),
        pltpu.VMEM((L,), jnp.float32),
        pltpu.SemaphoreType.DMA,
        pltpu.SemaphoreType.DMA,
        pltpu.SemaphoreType.DMA,
        pltpu.SemaphoreType.DMA,
    ],
    compiler_params=pltpu.CompilerParams(
        needs_layout_passes=False, use_tc_tiling_on_sc=True),
)
def _center_loss_kernel(x_hbm, labels_hbm, centers_hbm, out_hbm,
                        idx_v, c_v, x_v, part_v, gsem0, gsem1, xsem0, xsem1):
    gsems = (gsem0, gsem1)
    xsems = (xsem0, xsem1)
    wid = lax.axis_index("s") * NC + lax.axis_index("c")
    base = wid * BPW

    pltpu.sync_copy(labels_hbm.at[pl.ds(base, BPW)], idx_v)

    lane = lax.iota(jnp.int32, L)

    def make_group_body(pb):
        def group_body(g, tot):
            rows = g * L + lane
            accs = [jnp.zeros((L,), jnp.float32) for _ in range(4)]
            for f in range(FEAT_DIM):
                col = (lane + f) & (FEAT_DIM - 1)
                c = plsc.load_gather(c_v.at[pb], [rows, col])
                xv = plsc.load_gather(x_v.at[pb], [rows, col])
                d = c - xv
                accs[f % 4] = accs[f % 4] + d * d
            acc = (accs[0] + accs[1]) + (accs[2] + accs[3])
            acc = jnp.clip(acc, 1e-12, 1e12)
            return tot + acc
        return group_body

    def fire_chunk(ch):
        pb = ch % 2
        pltpu.async_copy(
            x_hbm.at[pl.ds(base + ch * CH, CH)], x_v.at[pb], xsems[pb])

        def fire(blk, _):
            vec = idx_v[pl.ds(ch * CH + blk * L, L)]
            for j in range(L):
                pltpu.async_copy(
                    centers_hbm.at[vec[j]], c_v.at[pb, blk * L + j],
                    gsems[pb])
            return 0

        lax.fori_loop(0, CH // L, fire, 0)

    tot = jnp.zeros((L,), jnp.float32)
    fire_chunk(0)
    for ch in range(NCH):
        pb = ch % 2
        if ch + 1 < NCH:
            fire_chunk(ch + 1)
        pltpu.make_async_copy(x_hbm.at[pl.ds(0, CH)], c_v.at[pb],
                              gsems[pb]).wait()
        pltpu.make_async_copy(x_hbm.at[pl.ds(0, CH)], x_v.at[pb],
                              xsems[pb]).wait()
        tot = lax.fori_loop(0, CH // L, make_group_body(pb), tot)
    part_v[...] = tot
    pltpu.sync_copy(part_v, out_hbm.at[wid])


def kernel(x, labels, centers):
    labels1 = labels.astype(jnp.int32)
    parts = _center_loss_kernel(x, labels1, centers)
    return jnp.sum(parts) / BATCH
